# baseline (speedup 1.0000x reference)
import jax
import jax.numpy as jnp
from jax.experimental import pallas as pl
from jax.experimental.pallas import tpu as pltpu

NEG = jnp.finfo(jnp.float32).min
_NS = 4


def _body(*refs):
    a21 = refs[:_NS]
    a12 = refs[_NS:2 * _NS]
    f2, f1, x1, x2, w1s, w1n, w2s, w2n, out1, out2 = refs[2 * _NS:]
    n = f2.shape[1]
    q = n // _NS

    m = None
    h = None
    s = None
    for i in range(_NS):
        f2i = f2[:, i * q:(i + 1) * q]
        f1i = f1[:, i * q:(i + 1) * q]
        ai = a21[i][...]
        mi = jnp.max(jnp.where(ai != 0, f2i, NEG), axis=1, keepdims=True)
        hi = jnp.max(ai, axis=1, keepdims=True)
        si = jnp.sum(jnp.where(a12[i][...] != 0, f1i, 0.0), axis=1,
                     keepdims=True)
        m = mi if m is None else jnp.maximum(m, mi)
        h = hi if h is None else jnp.maximum(h, hi)
        s = si if s is None else s + si

    scal1 = jnp.where(h > 0, m, 0.0)
    wsum1 = jnp.sum(w1n[...], axis=1)
    wsum2 = jnp.sum(w2n[...], axis=1)
    o1 = jnp.dot(x1[...], w1s[...].T, preferred_element_type=jnp.float32)
    o2 = jnp.dot(x2[...], w2s[...].T, preferred_element_type=jnp.float32)
    out1[...] = jnp.maximum(o1 + scal1 * wsum1[None, :], 0.0)
    out2[...] = jnp.maximum(o2 + s * wsum2[None, :], 0.0)


def kernel(node_feats1, node_feats2, adj_1to2, adj_2to1,
           W1_self, W1_neigh, W2_self, W2_neigh):
    n1, d_in = node_feats1.shape
    n2, _ = node_feats2.shape
    d_out = W1_self.shape[0]

    br = 256
    qc = n2 // _NS
    nr = n1 // br

    f2_row = node_feats2[:, 0].reshape(1, n2)
    f1_row = node_feats1[:, 0].reshape(1, n1)

    def col_spec(i):
        return pl.BlockSpec((br, qc), lambda r, i=i: (r, i))

    out1, out2 = pl.pallas_call(
        _body,
        grid=(nr,),
        in_specs=(
            [col_spec(i) for i in range(_NS)]
            + [col_spec(i) for i in range(_NS)]
            + [
                pl.BlockSpec((1, n2), lambda r: (0, 0)),
                pl.BlockSpec((1, n1), lambda r: (0, 0)),
                pl.BlockSpec((br, d_in), lambda r: (r, 0)),
                pl.BlockSpec((br, d_in), lambda r: (r, 0)),
                pl.BlockSpec((d_out, d_in), lambda r: (0, 0)),
                pl.BlockSpec((d_out, d_in), lambda r: (0, 0)),
                pl.BlockSpec((d_out, d_in), lambda r: (0, 0)),
                pl.BlockSpec((d_out, d_in), lambda r: (0, 0)),
            ]
        ),
        out_specs=[
            pl.BlockSpec((br, d_out), lambda r: (r, 0)),
            pl.BlockSpec((br, d_out), lambda r: (r, 0)),
        ],
        out_shape=[
            jax.ShapeDtypeStruct((n1, d_out), jnp.float32),
            jax.ShapeDtypeStruct((n2, d_out), jnp.float32),
        ],
        compiler_params=pltpu.CompilerParams(
            dimension_semantics=("parallel",),
        ),
    )(*([adj_2to1] * _NS), *([adj_1to2] * _NS), f2_row, f1_row,
      node_feats1, node_feats2, W1_self, W1_neigh, W2_self, W2_neigh)
    return out1, out2
